# Initial kernel scaffold; baseline (speedup 1.0000x reference)
import jax
import jax.numpy as jnp
from jax.experimental import pallas as pl


def kernel(grad_output, mask, sizes):
    raise NotImplementedError("write your pallas kernel here")


# speedup vs baseline: 45.8635x; 45.8635x over previous
import functools

import jax
import jax.numpy as jnp
from jax import lax
from jax.experimental import pallas as pl
from jax.experimental.pallas import tpu as pltpu
from jax.experimental.pallas import tpu_sc as plsc

NC = 2
NS = 16
NW = NC * NS


def _wid():
    return lax.axis_index("s") * NC + lax.axis_index("c")


def _mesh():
    return plsc.VectorSubcoreMesh(
        core_axis_name="c", subcore_axis_name="s", num_cores=NC, num_subcores=NS
    )


@functools.lru_cache(maxsize=None)
def _make_count(n, sb, interpret=False):
    c = n // NW
    nsb = c // sb
    assert nsb % 2 == 0

    @functools.partial(
        pl.kernel,
        out_type=jax.ShapeDtypeStruct((NW, 16), jnp.int32),
        mesh=_mesh(),
        compiler_params=pltpu.CompilerParams(needs_layout_passes=False),
        scratch_types=[
            pltpu.VMEM((sb,), jnp.int32),
            pltpu.VMEM((sb,), jnp.int32),
            pltpu.VMEM((16,), jnp.int32),
            pltpu.SemaphoreType.DMA,
            pltpu.SemaphoreType.DMA,
        ],
        interpret=interpret,
    )
    def count_k(m_hbm, out_hbm, mbuf0, mbuf1, cvec, sem0, sem1):
        w = _wid()
        base = w * c

        def half(g, mb, sem_this, mb_o, sem_other, acc):
            @pl.when(g + 1 < nsb)
            def _():
                off = pl.multiple_of(base + (g + 1) * sb, sb)
                pltpu.async_copy(m_hbm.at[pl.ds(off, sb)], mb_o, sem_other)

            pltpu.make_async_copy(m_hbm.at[pl.ds(0, sb)], mb, sem_this).wait()

            @pl.loop(0, sb // 16, init_carry=acc, unroll=8)
            def inner(i, a):
                return a + mb[pl.ds(i * 16, 16)]

            return inner

        pltpu.async_copy(m_hbm.at[pl.ds(pl.multiple_of(base, sb), sb)],
                         mbuf0, sem0)

        def outer(t, acc):
            acc = half(2 * t, mbuf0, sem0, mbuf1, sem1, acc)
            acc = half(2 * t + 1, mbuf1, sem1, mbuf0, sem0, acc)
            return acc

        acc = lax.fori_loop(0, nsb // 2, outer, jnp.zeros((16,), jnp.int32))
        cvec[...] = jnp.broadcast_to(jnp.sum(acc), (16,))
        pltpu.sync_copy(cvec, out_hbm.at[w])

    return count_k


@functools.lru_cache(maxsize=None)
def _make_scatter(n, sb, interpret=False):
    c = n // NW
    nsb = c // sb
    assert nsb % 2 == 0
    rbuf = 3 * sb
    maxr = 64
    npad = 64

    @functools.partial(
        pl.kernel,
        out_type=jax.ShapeDtypeStruct((n + npad,), jnp.float32),
        mesh=_mesh(),
        compiler_params=pltpu.CompilerParams(needs_layout_passes=False),
        scratch_types=[
            pltpu.VMEM((sb,), jnp.float32),
            pltpu.VMEM((sb,), jnp.float32),
            pltpu.VMEM((sb,), jnp.int32),
            pltpu.VMEM((sb,), jnp.int32),
            pltpu.VMEM((rbuf + 16,), jnp.float32),
            pltpu.VMEM((sb,), jnp.float32),
            pltpu.VMEM((maxr, 16), jnp.float32),
            pltpu.VMEM((64,), jnp.int32),
            pltpu.VMEM((16,), jnp.float32),
            pltpu.SemaphoreType.DMA,
            pltpu.SemaphoreType.DMA,
            pltpu.SemaphoreType.DMA,
            pltpu.SemaphoreType.DMA,
        ],
        interpret=interpret,
    )
    def scat_k(x_hbm, m_hbm, b_hbm, out_hbm,
               vbuf0, vbuf1, mbuf0, mbuf1, cbuf, zbuf, tbuf, bvm, dbuf,
               sema, semb, semf, semz):
        w = _wid()
        lane = lax.iota(jnp.int32, 16)
        pltpu.sync_copy(b_hbm, bvm)
        base_w = bvm[pl.ds(w, 16)][0]
        hi = bvm[pl.ds(NW, 16)]
        tot = hi[0]
        scf = lax.convert_element_type(hi[1], jnp.float32)

        s0 = base_w & 15
        outb0 = base_w - s0
        zf = jnp.where(w == NW - 1, (16 - (tot & 15)) & 15, 0)

        @pl.loop(0, sb // 16)
        def _(i):
            zbuf[pl.ds(i * 16, 16)] = jnp.broadcast_to(scf, (16,))

        nr_all = n >> 4
        r0 = (tot + 15) >> 4
        trows = nr_all - r0
        per = (trows + NW - 1) // NW
        rw = ((per + 255) // 256) * 256
        zlo = r0 + w * rw
        zhi = jnp.minimum(zlo + rw, nr_all)
        crow = jnp.maximum(zhi - zlo, 0)
        nb = crow >> 8
        zoff0 = zlo << 4

        @pl.loop(0, nb)
        def _(b):
            pltpu.async_copy(
                zbuf, out_hbm.at[pl.ds(pl.multiple_of(zoff0 + b * sb, 16),
                                       sb)], semz)

        rr = crow & 255
        nb2 = rr >> 4
        zoff1 = zoff0 + nb * sb

        @pl.loop(0, nb2)
        def _(i):
            pltpu.async_copy(
                zbuf.at[pl.ds(0, 256)],
                out_hbm.at[pl.ds(pl.multiple_of(zoff1 + i * 256, 16), 256)],
                semz)

        rr2 = rr & 15
        zoff2 = zoff1 + nb2 * 256

        @pl.loop(0, rr2)
        def _(i):
            pltpu.async_copy(
                zbuf.at[pl.ds(0, 16)],
                out_hbm.at[pl.ds(pl.multiple_of(zoff2 + i * 16, 16), 16)],
                semz)

        inb = w * c

        def flush(fc):
            @pl.when(fc >= 1)
            def _():
                pltpu.make_async_copy(out_hbm.at[pl.ds(0, sb)], zbuf,
                                      semf).wait()

            rs = lax.rem(fc, 3) * sb
            ob = outb0 + fc * sb

            @pl.when(fc == 0)
            def _():
                idx0 = jnp.where(lane >= s0, ob + lane, n + 16 + lane)
                tbuf[0] = cbuf[pl.ds(pl.multiple_of(rs, 16), 16)]
                pltpu.async_copy(tbuf.at[0], out_hbm.at[idx0], semf)
                pltpu.async_copy(
                    cbuf.at[pl.ds(pl.multiple_of(rs + 16, 16), sb - 16)],
                    out_hbm.at[pl.ds(pl.multiple_of(ob + 16, 16), sb - 16)],
                    semf)

            @pl.when(fc > 0)
            def _():
                pltpu.async_copy(
                    cbuf.at[pl.ds(pl.multiple_of(rs, 16), sb)],
                    out_hbm.at[pl.ds(pl.multiple_of(ob, 16), sb)],
                    semf)

        def compress(vb, mb, carry):
            @pl.loop(0, sb // 16, init_carry=carry, unroll=8)
            def inner(i, cr):
                wpos, u = cr
                mv = mb[pl.ds(i * 16, 16)]
                xv = vb[pl.ds(i * 16, 16)] + scf
                mk = mv != 0
                plsc.store_compressed(cbuf.at[pl.ds(wpos, 16)], xv, mask=mk)
                cnt = plsc.all_reduce_population_count(mk)[0]
                w2 = wpos + cnt

                @pl.when(w2 > rbuf - 16)
                def _():
                    cbuf[pl.ds(0, 16)] = cbuf[pl.ds(rbuf, 16)]

                w3 = jnp.where(w2 >= rbuf, w2 - rbuf, w2)
                return (w3, u + cnt)

            return inner

        def half(g, vb, mb, sem_this, vb_o, mb_o, sem_other, carry):
            wpos, u, fc = carry

            @pl.when(g + 1 < nsb)
            def _():
                off = pl.multiple_of(inb + (g + 1) * sb, sb)
                pltpu.async_copy(x_hbm.at[pl.ds(off, sb)], vb_o, sem_other)
                pltpu.async_copy(m_hbm.at[pl.ds(off, sb)], mb_o, sem_other)

            pltpu.make_async_copy(x_hbm.at[pl.ds(0, sb)], vb, sem_this).wait()
            pltpu.make_async_copy(m_hbm.at[pl.ds(0, sb)], mb, sem_this).wait()

            wpos, u = compress(vb, mb, (wpos, u))
            do = u >= sb

            @pl.when(do)
            def _():
                flush(fc)

            u = jnp.where(do, u - sb, u)
            fc = jnp.where(do, fc + 1, fc)
            return (wpos, u, fc)

        off0 = pl.multiple_of(inb, sb)
        pltpu.async_copy(x_hbm.at[pl.ds(off0, sb)], vbuf0, sema)
        pltpu.async_copy(m_hbm.at[pl.ds(off0, sb)], mbuf0, sema)

        def outer(t, carry):
            carry = half(2 * t, vbuf0, mbuf0, sema, vbuf1, mbuf1, semb, carry)
            carry = half(2 * t + 1, vbuf1, mbuf1, semb, vbuf0, mbuf0, sema,
                         carry)
            return carry

        wpos, u, fc = lax.fori_loop(
            0, nsb // 2, outer, (s0, s0, jnp.int32(0)))

        @pl.when(fc >= 1)
        def _():
            pltpu.make_async_copy(out_hbm.at[pl.ds(0, sb)], zbuf, semf).wait()

        startab = fc * sb
        endab = startab + u
        zend = endab + zf
        nrt = (u + zf + 15) >> 4

        @pl.loop(0, nrt)
        def _(r):
            slot = r & (maxr - 1)
            ab0 = startab + r * 16
            pr = lax.rem(ab0, rbuf)
            v = cbuf[pl.ds(pl.multiple_of(pr, 16), 16)]
            ab = ab0 + lane
            vals = jnp.where(ab < endab, v, jnp.broadcast_to(scf, (16,)))
            valid = (ab >= s0) & (ab < zend)
            idx = jnp.where(valid, outb0 + ab, n + 16 + lane)
            tbuf[slot] = vals
            pltpu.async_copy(tbuf.at[slot], out_hbm.at[idx], sema)

            @pl.when((slot == maxr - 1) | (r == nrt - 1))
            def _():
                @pl.loop(0, slot + 1)
                def _(j):
                    pltpu.make_async_copy(out_hbm.at[pl.ds(0, 16)], dbuf,
                                          sema).wait()

        @pl.loop(0, nb)
        def _(b):
            pltpu.make_async_copy(out_hbm.at[pl.ds(0, sb)], zbuf, semz).wait()

        @pl.loop(0, nb2)
        def _(i):
            pltpu.make_async_copy(out_hbm.at[pl.ds(0, 256)],
                                  zbuf.at[pl.ds(0, 256)], semz).wait()

        @pl.loop(0, rr2)
        def _(i):
            pltpu.make_async_copy(out_hbm.at[pl.ds(0, 16)], dbuf, semz).wait()

    return scat_k


def _compact(flat, mi, sizes_i32, n, sb, interpret=False):
    counts2d = _make_count(n, sb, interpret)(mi)
    counts = counts2d[:, 0]
    bases = jnp.concatenate(
        [jnp.zeros((1,), jnp.int32), jnp.cumsum(counts, dtype=jnp.int32)])
    extra = sizes_i32 - jnp.int32(n)
    bvec = jnp.concatenate(
        [bases, jnp.reshape(extra, (1,)), jnp.zeros((30,), jnp.int32)])
    outp = _make_scatter(n, sb, interpret)(flat, mi, bvec)
    return outp[:n]


def kernel(grad_output, mask, sizes):
    n = grad_output.size
    flat = grad_output.reshape(-1)
    mi = mask.reshape(-1).astype(jnp.int32)
    sizes_i32 = jnp.asarray(sizes, jnp.int32)
    return _compact(flat, mi, sizes_i32, n, 4096)
